# baseline (speedup 1.0000x reference)
import functools

import jax
import jax.numpy as jnp
from jax import lax
from jax.experimental import pallas as pl
from jax.experimental.pallas import tpu as pltpu
from jax.experimental.pallas import tpu_sc as plsc

_N = 10000
_E = 320000
_D = 128
_G = 64
_T = 128
_L = 5

_NC = 2
_NS = 16
_NW = _NC * _NS
_K = 128
_EPW = 10240
_NB = _EPW // _K
_CB = 8
_NCH = _NB // _CB
_NP = 10112
_ZR = 128

_R = 1264
_NBLK = _NP // _R

def _sc_edge_agg_body(h_hbm, srcb_hbm, dstb_hbm, out_hbm,
                      scb0, scb1, dcb0, dcb1, buf0, buf1, agg_sh,
                      sem0, sem1, isem0, isem1):
    c = lax.axis_index("c")
    s = lax.axis_index("s")
    wid = c * _NS + s
    base = s * 640
    nz = jnp.where(s == _NS - 1, 4, 5)

    def _z(i, carry):
        buf0[i // 8, pl.ds((i % 8) * 16, 16)] = jnp.zeros((16,), jnp.float32)
        return carry
    lax.fori_loop(0, _ZR * 8, _z, 0)

    def _zs(t, carry):
        pltpu.async_copy(buf0, agg_sh.at[pl.ds(base + t * _ZR, _ZR)], sem0)
        return carry
    lax.fori_loop(0, nz, _zs, 0)

    def _zw(t, carry):
        pltpu.make_async_copy(buf0, agg_sh.at[pl.ds(base + t * _ZR, _ZR)],
                              sem0).wait()
        return carry
    lax.fori_loop(0, nz, _zw, 0)
    plsc.subcore_barrier()

    bufs = (buf0, buf1)
    sems = (sem0, sem1)

    def _chunk(scb, dcb):
        pltpu.async_copy(h_hbm.at[scb.at[0]], buf0, sem0)
        for b in range(_CB):
            if b + 1 < _CB:
                pltpu.async_copy(h_hbm.at[scb.at[b + 1]],
                                 bufs[(b + 1) & 1], sems[(b + 1) & 1])
            pltpu.make_async_copy(h_hbm.at[scb.at[b]],
                                  bufs[b & 1], sems[b & 1]).wait()
            pltpu.sync_copy(bufs[b & 1], agg_sh.at[dcb.at[b]], add=True)

    def _idx_start(g, scb, dcb, isem):
        pltpu.async_copy(srcb_hbm.at[wid, pl.ds(g * _CB, _CB)], scb, isem)
        pltpu.async_copy(dstb_hbm.at[wid, pl.ds(g * _CB, _CB)], dcb, isem)

    def _idx_wait(g, scb, dcb, isem):
        pltpu.make_async_copy(srcb_hbm.at[wid, pl.ds(g * _CB, _CB)], scb,
                              isem).wait()
        pltpu.make_async_copy(dstb_hbm.at[wid, pl.ds(g * _CB, _CB)], dcb,
                              isem).wait()

    _idx_start(0, scb0, dcb0, isem0)

    def _pair(v, carry):
        g0 = 2 * v
        _idx_start(g0 + 1, scb1, dcb1, isem1)
        _idx_wait(g0, scb0, dcb0, isem0)
        _chunk(scb0, dcb0)

        @pl.when(g0 + 2 < _NCH)
        def _():
            _idx_start(g0 + 2, scb0, dcb0, isem0)
        _idx_wait(g0 + 1, scb1, dcb1, isem1)
        _chunk(scb1, dcb1)
        return carry
    lax.fori_loop(0, _NCH // 2, _pair, 0)
    plsc.subcore_barrier()

    def _wo(t, carry):
        r0 = base + t * _ZR
        pltpu.async_copy(agg_sh.at[pl.ds(r0, _ZR)], out_hbm.at[c, pl.ds(r0, _ZR)],
                         sem0)
        return carry
    lax.fori_loop(0, nz, _wo, 0)

    def _ww(t, carry):
        r0 = base + t * _ZR
        pltpu.make_async_copy(agg_sh.at[pl.ds(r0, _ZR)],
                              out_hbm.at[c, pl.ds(r0, _ZR)], sem0).wait()
        return carry
    lax.fori_loop(0, nz, _ww, 0)


@functools.cache
def _sc_edge_agg():
    mesh = plsc.VectorSubcoreMesh(core_axis_name="c", subcore_axis_name="s",
                                  num_cores=_NC, num_subcores=_NS)
    return pl.kernel(
        _sc_edge_agg_body,
        out_type=jax.ShapeDtypeStruct((_NC, _NP, _D), jnp.float32),
        mesh=mesh,
        scratch_types=[
            pltpu.VMEM((_CB, _K), jnp.int32),
            pltpu.VMEM((_CB, _K), jnp.int32),
            pltpu.VMEM((_CB, _K), jnp.int32),
            pltpu.VMEM((_CB, _K), jnp.int32),
            pltpu.VMEM((_K, _D), jnp.float32),
            pltpu.VMEM((_K, _D), jnp.float32),
            pltpu.VMEM_SHARED((_NP, _D), jnp.float32),
            pltpu.SemaphoreType.DMA,
            pltpu.SemaphoreType.DMA,
            pltpu.SemaphoreType.DMA,
            pltpu.SemaphoreType.DMA,
        ],
    )


def _dot(a, b, dims):
    return lax.dot_general(a, b, (dims, ((), ())),
                           precision=lax.Precision.HIGHEST,
                           preferred_element_type=jnp.float32)


def _tc_layer_body(scale_ref, h_ref, p_ref, w_ref, b_ref, o_ref):
    z = scale_ref[0, 0] * h_ref[...] + p_ref[0] + p_ref[1]
    y = jnp.maximum(_dot(z, w_ref[...], ((1,), (0,))) + b_ref[...], 0.0)
    rid = pl.program_id(0) * _R + lax.broadcasted_iota(jnp.int32, (_R, _D), 0)
    o_ref[...] = jnp.where(rid < _N, y, 0.0)


_tc_layer = pl.pallas_call(
    _tc_layer_body,
    grid=(_NBLK,),
    in_specs=[
        pl.BlockSpec(memory_space=pltpu.SMEM),
        pl.BlockSpec((_R, _D), lambda i: (i, 0)),
        pl.BlockSpec((_NC, _R, _D), lambda i: (0, i, 0)),
        pl.BlockSpec((_D, _D), lambda i: (0, 0)),
        pl.BlockSpec((1, _D), lambda i: (0, 0)),
    ],
    out_specs=pl.BlockSpec((_R, _D), lambda i: (i, 0)),
    out_shape=jax.ShapeDtypeStruct((_NP, _D), jnp.float32),
)


def _tc_final_body(scale_ref, h_ref, p_ref, w4_ref, b4_ref, wn_ref, bn_ref,
                   wo_ref, bo_ref, bat_ref, o_ref, sums_ref, cnt_ref):
    i = pl.program_id(0)

    @pl.when(i == 0)
    def _():
        sums_ref[...] = jnp.zeros((_G, _D), jnp.float32)
        cnt_ref[...] = jnp.zeros((_G, _D), jnp.float32)

    z = scale_ref[0, 0] * h_ref[...] + p_ref[0] + p_ref[1]
    h5 = jnp.maximum(_dot(z, w4_ref[...], ((1,), (0,))) + b4_ref[...], 0.0)
    h6 = jnp.maximum(_dot(h5, wn_ref[...], ((1,), (0,))) + bn_ref[...], 0.0)
    gids = lax.broadcasted_iota(jnp.int32, (_R, _G), 1).astype(jnp.float32)
    oh = (bat_ref[...] == gids)
    oh = oh.astype(jnp.float32)
    sums_ref[...] += _dot(oh, h6, ((0,), (0,)))
    cnt_ref[...] += jnp.sum(oh, axis=0)[:, None]

    @pl.when(i == _NBLK - 1)
    def _():
        hg = sums_ref[...] / jnp.maximum(cnt_ref[...], 1.0)
        o_ref[...] = _dot(hg, wo_ref[...], ((1,), (0,))) + bo_ref[...]


_tc_final = pl.pallas_call(
    _tc_final_body,
    grid=(_NBLK,),
    in_specs=[
        pl.BlockSpec(memory_space=pltpu.SMEM),
        pl.BlockSpec((_R, _D), lambda i: (i, 0)),
        pl.BlockSpec((_NC, _R, _D), lambda i: (0, i, 0)),
        pl.BlockSpec((_D, _D), lambda i: (0, 0)),
        pl.BlockSpec((1, _D), lambda i: (0, 0)),
        pl.BlockSpec((_D, _D), lambda i: (0, 0)),
        pl.BlockSpec((1, _D), lambda i: (0, 0)),
        pl.BlockSpec((_D, _T), lambda i: (0, 0)),
        pl.BlockSpec((1, _T), lambda i: (0, 0)),
        pl.BlockSpec((_R, 1), lambda i: (i, 0)),
    ],
    out_specs=pl.BlockSpec((_G, _T), lambda i: (0, 0)),
    out_shape=jax.ShapeDtypeStruct((_G, _T), jnp.float32),
    scratch_shapes=[
        pltpu.VMEM((_G, _D), jnp.float32),
        pltpu.VMEM((_G, _D), jnp.float32),
    ],
)


def kernel(x, edge_index, batch, Wl, bl, eps, Wn2n, bn2n, Wout, bout):
    pad = _EPW - _E // _NW
    dummy_src = _N + (jnp.arange(pad, dtype=jnp.int32) % (_NP - _N))
    w_ids = jnp.arange(_NW, dtype=jnp.int32)[:, None]
    dummy_dst = (w_ids * 997 + jnp.arange(pad, dtype=jnp.int32)[None, :] * 131) % _NP
    src_b = jnp.concatenate(
        [edge_index[0].reshape(_NW, _E // _NW),
         jnp.broadcast_to(dummy_src, (_NW, pad))], axis=1).reshape(_NW, _NB, _K)
    dst_b = jnp.concatenate(
        [edge_index[1].reshape(_NW, _E // _NW),
         dummy_dst], axis=1).reshape(_NW, _NB, _K)
    batch_f = jnp.concatenate(
        [batch.astype(jnp.float32),
         jnp.full((_NP - _N,), -1.0, jnp.float32)]).reshape(_NP, 1)
    h = jnp.concatenate([x, jnp.zeros((_NP - _N, _D), jnp.float32)], axis=0)
    for l in range(_L):
        parts = _sc_edge_agg()(h, src_b, dst_b)
        scale = (1.0 + eps[l]).reshape(1, 1)
        b_l = bl[l].reshape(1, _D)
        if l < _L - 1:
            h = _tc_layer(scale, h, parts, Wl[l], b_l)
        else:
            out = _tc_final(scale, h, parts, Wl[l], b_l,
                            Wn2n, bn2n.reshape(1, _D),
                            Wout, bout.reshape(1, _T), batch_f)
    return out


# speedup vs baseline: 1.1402x; 1.1402x over previous
import functools

import jax
import jax.numpy as jnp
from jax import lax
from jax.experimental import pallas as pl
from jax.experimental.pallas import tpu as pltpu
from jax.experimental.pallas import tpu_sc as plsc

_N = 10000
_E = 320000
_D = 128
_G = 64
_T = 128
_L = 5

_NC = 2
_NS = 16
_NW = _NC * _NS
_K = 64
_EPW = 10240
_NB = _EPW // _K
_CB = 8
_NCH = _NB // _CB
_NBODY = _NCH // 2
_NP = 10112
_ZR = 128

_R = 1264
_NBLK = _NP // _R

def _sc_edge_agg_body(h_hbm, srcb_hbm, dstb_hbm, out_hbm,
                      scb0, scb1, dcb0, dcb1, b0, b1, b2, b3, agg_sh,
                      g0, g1, g2, g3, s0, s1, s2, s3, isem0, isem1):
    c = lax.axis_index("c")
    s = lax.axis_index("s")
    wid = c * _NS + s
    base = s * 640
    nz = jnp.where(s == _NS - 1, 4, 5)
    bufs = (b0, b1, b2, b3)
    gsems = (g0, g1, g2, g3)
    ssems = (s0, s1, s2, s3)

    def _z(i, carry):
        b0[i // 8, pl.ds((i % 8) * 16, 16)] = jnp.zeros((16,), jnp.float32)
        return carry
    lax.fori_loop(0, _K * 8, _z, 0)

    def _zs(t, carry):
        pltpu.async_copy(b0, agg_sh.at[pl.ds(base + t * _K, _K)], g0)
        return carry
    lax.fori_loop(0, 2 * nz, _zs, 0)

    def _zw(t, carry):
        pltpu.make_async_copy(b0, agg_sh.at[pl.ds(base + t * _K, _K)],
                              g0).wait()
        return carry
    lax.fori_loop(0, 2 * nz, _zw, 0)
    plsc.subcore_barrier()

    def _idx_start(g, scb, dcb, isem):
        pltpu.async_copy(srcb_hbm.at[wid, pl.ds(g * _CB, _CB)], scb, isem)
        pltpu.async_copy(dstb_hbm.at[wid, pl.ds(g * _CB, _CB)], dcb, isem)

    def _idx_wait(g, scb, dcb, isem):
        pltpu.make_async_copy(srcb_hbm.at[wid, pl.ds(g * _CB, _CB)], scb,
                              isem).wait()
        pltpu.make_async_copy(dstb_hbm.at[wid, pl.ds(g * _CB, _CB)], dcb,
                              isem).wait()

    _idx_start(0, scb0, dcb0, isem0)
    _idx_start(1, scb1, dcb1, isem1)
    _idx_wait(0, scb0, dcb0, isem0)
    pltpu.async_copy(h_hbm.at[scb0.at[0]], b0, g0)
    pltpu.async_copy(h_hbm.at[scb0.at[1]], b1, g1)
    for q in (2, 3):
        pltpu.async_copy(b0, agg_sh.at[dcb0.at[0]], ssems[q], add=True)

    def _body(v, carry):
        for j in range(16):
            q = j & 3
            r = (j + 2) & 3
            scb_j, dcb_j = (scb0, dcb0) if j < 8 else (scb1, dcb1)
            if j == 2:
                @pl.when(v > 0)
                def _():
                    _idx_start(2 * v + 1, scb1, dcb1, isem1)
            if j == 6:
                _idx_wait(2 * v + 1, scb1, dcb1, isem1)
            if j == 10:
                @pl.when(v + 1 < _NBODY)
                def _():
                    _idx_start(2 * v + 2, scb0, dcb0, isem0)
            if j == 13:
                @pl.when(v + 1 < _NBODY)
                def _():
                    _idx_wait(2 * v + 2, scb0, dcb0, isem0)
            pltpu.make_async_copy(b0, agg_sh.at[dcb0.at[0]], ssems[r]).wait()
            if j + 2 < 16:
                scb_n, dcb_n = (scb0, dcb0) if j + 2 < 8 else (scb1, dcb1)
                pltpu.async_copy(h_hbm.at[scb_n.at[(j + 2) & 7]],
                                 bufs[r], gsems[r])
            else:
                @pl.when(v + 1 < _NBODY)
                def _():
                    pltpu.async_copy(h_hbm.at[scb0.at[(j + 2) & 7]],
                                     bufs[r], gsems[r])
            pltpu.make_async_copy(h_hbm.at[scb_j.at[j & 7]],
                                  bufs[q], gsems[q]).wait()
            pltpu.async_copy(bufs[q], agg_sh.at[dcb_j.at[j & 7]],
                             ssems[q], add=True)
        return carry
    lax.fori_loop(0, _NBODY, _body, 0)

    for q in (2, 3):
        pltpu.make_async_copy(b0, agg_sh.at[dcb0.at[0]], ssems[q]).wait()
    plsc.subcore_barrier()

    def _wo(t, carry):
        r0 = base + t * _ZR
        pltpu.async_copy(agg_sh.at[pl.ds(r0, _ZR)], out_hbm.at[c, pl.ds(r0, _ZR)],
                         g0)
        return carry
    lax.fori_loop(0, nz, _wo, 0)

    def _ww(t, carry):
        r0 = base + t * _ZR
        pltpu.make_async_copy(agg_sh.at[pl.ds(r0, _ZR)],
                              out_hbm.at[c, pl.ds(r0, _ZR)], g0).wait()
        return carry
    lax.fori_loop(0, nz, _ww, 0)


@functools.cache
def _sc_edge_agg():
    mesh = plsc.VectorSubcoreMesh(core_axis_name="c", subcore_axis_name="s",
                                  num_cores=_NC, num_subcores=_NS)
    return pl.kernel(
        _sc_edge_agg_body,
        out_type=jax.ShapeDtypeStruct((_NC, _NP, _D), jnp.float32),
        mesh=mesh,
        scratch_types=[
            pltpu.VMEM((_CB, _K), jnp.int32),
            pltpu.VMEM((_CB, _K), jnp.int32),
            pltpu.VMEM((_CB, _K), jnp.int32),
            pltpu.VMEM((_CB, _K), jnp.int32),
            pltpu.VMEM((_K, _D), jnp.float32),
            pltpu.VMEM((_K, _D), jnp.float32),
            pltpu.VMEM((_K, _D), jnp.float32),
            pltpu.VMEM((_K, _D), jnp.float32),
            pltpu.VMEM_SHARED((_NP, _D), jnp.float32),
            pltpu.SemaphoreType.DMA,
            pltpu.SemaphoreType.DMA,
            pltpu.SemaphoreType.DMA,
            pltpu.SemaphoreType.DMA,
            pltpu.SemaphoreType.DMA,
            pltpu.SemaphoreType.DMA,
            pltpu.SemaphoreType.DMA,
            pltpu.SemaphoreType.DMA,
            pltpu.SemaphoreType.DMA,
            pltpu.SemaphoreType.DMA,
        ],
    )


def _dot(a, b, dims):
    return lax.dot_general(a, b, (dims, ((), ())),
                           precision=lax.Precision.HIGHEST,
                           preferred_element_type=jnp.float32)


def _tc_layer_body(scale_ref, h_ref, p_ref, w_ref, b_ref, o_ref):
    z = scale_ref[0, 0] * h_ref[...] + p_ref[0] + p_ref[1]
    y = jnp.maximum(_dot(z, w_ref[...], ((1,), (0,))) + b_ref[...], 0.0)
    rid = pl.program_id(0) * _R + lax.broadcasted_iota(jnp.int32, (_R, _D), 0)
    o_ref[...] = jnp.where(rid < _N, y, 0.0)


_tc_layer = pl.pallas_call(
    _tc_layer_body,
    grid=(_NBLK,),
    in_specs=[
        pl.BlockSpec(memory_space=pltpu.SMEM),
        pl.BlockSpec((_R, _D), lambda i: (i, 0)),
        pl.BlockSpec((_NC, _R, _D), lambda i: (0, i, 0)),
        pl.BlockSpec((_D, _D), lambda i: (0, 0)),
        pl.BlockSpec((1, _D), lambda i: (0, 0)),
    ],
    out_specs=pl.BlockSpec((_R, _D), lambda i: (i, 0)),
    out_shape=jax.ShapeDtypeStruct((_NP, _D), jnp.float32),
)


def _tc_final_body(scale_ref, h_ref, p_ref, w4_ref, b4_ref, wn_ref, bn_ref,
                   wo_ref, bo_ref, bat_ref, o_ref, sums_ref, cnt_ref):
    i = pl.program_id(0)

    @pl.when(i == 0)
    def _():
        sums_ref[...] = jnp.zeros((_G, _D), jnp.float32)
        cnt_ref[...] = jnp.zeros((_G, _D), jnp.float32)

    z = scale_ref[0, 0] * h_ref[...] + p_ref[0] + p_ref[1]
    h5 = jnp.maximum(_dot(z, w4_ref[...], ((1,), (0,))) + b4_ref[...], 0.0)
    h6 = jnp.maximum(_dot(h5, wn_ref[...], ((1,), (0,))) + bn_ref[...], 0.0)
    gids = lax.broadcasted_iota(jnp.int32, (_R, _G), 1).astype(jnp.float32)
    oh = (bat_ref[...] == gids)
    oh = oh.astype(jnp.float32)
    sums_ref[...] += _dot(oh, h6, ((0,), (0,)))
    cnt_ref[...] += jnp.sum(oh, axis=0)[:, None]

    @pl.when(i == _NBLK - 1)
    def _():
        hg = sums_ref[...] / jnp.maximum(cnt_ref[...], 1.0)
        o_ref[...] = _dot(hg, wo_ref[...], ((1,), (0,))) + bo_ref[...]


_tc_final = pl.pallas_call(
    _tc_final_body,
    grid=(_NBLK,),
    in_specs=[
        pl.BlockSpec(memory_space=pltpu.SMEM),
        pl.BlockSpec((_R, _D), lambda i: (i, 0)),
        pl.BlockSpec((_NC, _R, _D), lambda i: (0, i, 0)),
        pl.BlockSpec((_D, _D), lambda i: (0, 0)),
        pl.BlockSpec((1, _D), lambda i: (0, 0)),
        pl.BlockSpec((_D, _D), lambda i: (0, 0)),
        pl.BlockSpec((1, _D), lambda i: (0, 0)),
        pl.BlockSpec((_D, _T), lambda i: (0, 0)),
        pl.BlockSpec((1, _T), lambda i: (0, 0)),
        pl.BlockSpec((_R, 1), lambda i: (i, 0)),
    ],
    out_specs=pl.BlockSpec((_G, _T), lambda i: (0, 0)),
    out_shape=jax.ShapeDtypeStruct((_G, _T), jnp.float32),
    scratch_shapes=[
        pltpu.VMEM((_G, _D), jnp.float32),
        pltpu.VMEM((_G, _D), jnp.float32),
    ],
)


def kernel(x, edge_index, batch, Wl, bl, eps, Wn2n, bn2n, Wout, bout):
    pad = _EPW - _E // _NW
    dummy_src = _N + (jnp.arange(pad, dtype=jnp.int32) % (_NP - _N))
    w_ids = jnp.arange(_NW, dtype=jnp.int32)[:, None]
    dummy_dst = (w_ids * 997 + jnp.arange(pad, dtype=jnp.int32)[None, :] * 131) % _NP
    src_b = jnp.concatenate(
        [edge_index[0].reshape(_NW, _E // _NW),
         jnp.broadcast_to(dummy_src, (_NW, pad))], axis=1).reshape(_NW, _NB, _K)
    dst_b = jnp.concatenate(
        [edge_index[1].reshape(_NW, _E // _NW),
         dummy_dst], axis=1).reshape(_NW, _NB, _K)
    batch_f = jnp.concatenate(
        [batch.astype(jnp.float32),
         jnp.full((_NP - _N,), -1.0, jnp.float32)]).reshape(_NP, 1)
    h = jnp.concatenate([x, jnp.zeros((_NP - _N, _D), jnp.float32)], axis=0)
    for l in range(_L):
        parts = _sc_edge_agg()(h, src_b, dst_b)
        scale = (1.0 + eps[l]).reshape(1, 1)
        b_l = bl[l].reshape(1, _D)
        if l < _L - 1:
            h = _tc_layer(scale, h, parts, Wl[l], b_l)
        else:
            out = _tc_final(scale, h, parts, Wl[l], b_l,
                            Wn2n, bn2n.reshape(1, _D),
                            Wout, bout.reshape(1, _T), batch_f)
    return out


# speedup vs baseline: 1.1573x; 1.0150x over previous
import functools

import jax
import jax.numpy as jnp
from jax import lax
from jax.experimental import pallas as pl
from jax.experimental.pallas import tpu as pltpu
from jax.experimental.pallas import tpu_sc as plsc

_N = 10000
_E = 320000
_D = 128
_G = 64
_T = 128
_L = 5

_NC = 2
_NS = 16
_NW = _NC * _NS
_K = 64
_EPW = 10240
_NB = _EPW // _K
_CB = 8
_NCH = _NB // _CB
_NBODY = _NCH // 2
_NP = 10112
_ZR = 128

_R = 2528
_NBLK = _NP // _R

def _sc_edge_agg_body(h_hbm, srcb_hbm, dstb_hbm, out_hbm,
                      scb0, scb1, dcb0, dcb1, b0, b1, b2, b3, agg_sh,
                      g0, g1, g2, g3, s0, s1, s2, s3, isem0, isem1):
    c = lax.axis_index("c")
    s = lax.axis_index("s")
    wid = c * _NS + s
    base = s * 640
    nz = jnp.where(s == _NS - 1, 4, 5)
    bufs = (b0, b1, b2, b3)
    gsems = (g0, g1, g2, g3)
    ssems = (s0, s1, s2, s3)

    def _z(i, carry):
        b0[i // 8, pl.ds((i % 8) * 16, 16)] = jnp.zeros((16,), jnp.float32)
        return carry
    lax.fori_loop(0, _K * 8, _z, 0)

    def _zs(t, carry):
        pltpu.async_copy(b0, agg_sh.at[pl.ds(base + t * _K, _K)], g0)
        return carry
    lax.fori_loop(0, 2 * nz, _zs, 0)

    def _zw(t, carry):
        pltpu.make_async_copy(b0, agg_sh.at[pl.ds(base + t * _K, _K)],
                              g0).wait()
        return carry
    lax.fori_loop(0, 2 * nz, _zw, 0)
    plsc.subcore_barrier()

    def _idx_start(g, scb, dcb, isem):
        pltpu.async_copy(srcb_hbm.at[wid, pl.ds(g * _CB, _CB)], scb, isem)
        pltpu.async_copy(dstb_hbm.at[wid, pl.ds(g * _CB, _CB)], dcb, isem)

    def _idx_wait(g, scb, dcb, isem):
        pltpu.make_async_copy(srcb_hbm.at[wid, pl.ds(g * _CB, _CB)], scb,
                              isem).wait()
        pltpu.make_async_copy(dstb_hbm.at[wid, pl.ds(g * _CB, _CB)], dcb,
                              isem).wait()

    _idx_start(0, scb0, dcb0, isem0)
    _idx_start(1, scb1, dcb1, isem1)
    _idx_wait(0, scb0, dcb0, isem0)
    pltpu.async_copy(h_hbm.at[scb0.at[0]], b0, g0)
    pltpu.async_copy(h_hbm.at[scb0.at[1]], b1, g1)
    for q in (2, 3):
        pltpu.async_copy(b0, agg_sh.at[dcb0.at[0]], ssems[q], add=True)

    def _body(v, carry):
        for j in range(16):
            q = j & 3
            r = (j + 2) & 3
            scb_j, dcb_j = (scb0, dcb0) if j < 8 else (scb1, dcb1)
            if j == 2:
                @pl.when(v > 0)
                def _():
                    _idx_start(2 * v + 1, scb1, dcb1, isem1)
            if j == 6:
                _idx_wait(2 * v + 1, scb1, dcb1, isem1)
            if j == 10:
                @pl.when(v + 1 < _NBODY)
                def _():
                    _idx_start(2 * v + 2, scb0, dcb0, isem0)
            if j == 13:
                @pl.when(v + 1 < _NBODY)
                def _():
                    _idx_wait(2 * v + 2, scb0, dcb0, isem0)
            pltpu.make_async_copy(b0, agg_sh.at[dcb0.at[0]], ssems[r]).wait()
            if j + 2 < 16:
                scb_n, dcb_n = (scb0, dcb0) if j + 2 < 8 else (scb1, dcb1)
                pltpu.async_copy(h_hbm.at[scb_n.at[(j + 2) & 7]],
                                 bufs[r], gsems[r])
            else:
                @pl.when(v + 1 < _NBODY)
                def _():
                    pltpu.async_copy(h_hbm.at[scb0.at[(j + 2) & 7]],
                                     bufs[r], gsems[r])
            pltpu.make_async_copy(h_hbm.at[scb_j.at[j & 7]],
                                  bufs[q], gsems[q]).wait()
            pltpu.async_copy(bufs[q], agg_sh.at[dcb_j.at[j & 7]],
                             ssems[q], add=True)
        return carry
    lax.fori_loop(0, _NBODY, _body, 0)

    for q in (2, 3):
        pltpu.make_async_copy(b0, agg_sh.at[dcb0.at[0]], ssems[q]).wait()
    plsc.subcore_barrier()

    def _wo(t, carry):
        r0 = base + t * _ZR
        pltpu.async_copy(agg_sh.at[pl.ds(r0, _ZR)], out_hbm.at[c, pl.ds(r0, _ZR)],
                         g0)
        return carry
    lax.fori_loop(0, nz, _wo, 0)

    def _ww(t, carry):
        r0 = base + t * _ZR
        pltpu.make_async_copy(agg_sh.at[pl.ds(r0, _ZR)],
                              out_hbm.at[c, pl.ds(r0, _ZR)], g0).wait()
        return carry
    lax.fori_loop(0, nz, _ww, 0)


@functools.cache
def _sc_edge_agg():
    mesh = plsc.VectorSubcoreMesh(core_axis_name="c", subcore_axis_name="s",
                                  num_cores=_NC, num_subcores=_NS)
    return pl.kernel(
        _sc_edge_agg_body,
        out_type=jax.ShapeDtypeStruct((_NC, _NP, _D), jnp.float32),
        mesh=mesh,
        scratch_types=[
            pltpu.VMEM((_CB, _K), jnp.int32),
            pltpu.VMEM((_CB, _K), jnp.int32),
            pltpu.VMEM((_CB, _K), jnp.int32),
            pltpu.VMEM((_CB, _K), jnp.int32),
            pltpu.VMEM((_K, _D), jnp.float32),
            pltpu.VMEM((_K, _D), jnp.float32),
            pltpu.VMEM((_K, _D), jnp.float32),
            pltpu.VMEM((_K, _D), jnp.float32),
            pltpu.VMEM_SHARED((_NP, _D), jnp.float32),
            pltpu.SemaphoreType.DMA,
            pltpu.SemaphoreType.DMA,
            pltpu.SemaphoreType.DMA,
            pltpu.SemaphoreType.DMA,
            pltpu.SemaphoreType.DMA,
            pltpu.SemaphoreType.DMA,
            pltpu.SemaphoreType.DMA,
            pltpu.SemaphoreType.DMA,
            pltpu.SemaphoreType.DMA,
            pltpu.SemaphoreType.DMA,
        ],
    )


def _dot(a, b, dims):
    return lax.dot_general(a, b, (dims, ((), ())),
                           precision=lax.Precision.HIGHEST,
                           preferred_element_type=jnp.float32)


def _tc_layer_body(scale_ref, h_ref, p_ref, w_ref, b_ref, o_ref):
    z = scale_ref[0, 0] * h_ref[...] + p_ref[0] + p_ref[1]
    y = jnp.maximum(_dot(z, w_ref[...], ((1,), (0,))) + b_ref[...], 0.0)
    rid = pl.program_id(0) * _R + lax.broadcasted_iota(jnp.int32, (_R, _D), 0)
    o_ref[...] = jnp.where(rid < _N, y, 0.0)


_tc_layer = pl.pallas_call(
    _tc_layer_body,
    grid=(_NBLK,),
    in_specs=[
        pl.BlockSpec(memory_space=pltpu.SMEM),
        pl.BlockSpec((_R, _D), lambda i: (i, 0)),
        pl.BlockSpec((_NC, _R, _D), lambda i: (0, i, 0)),
        pl.BlockSpec((_D, _D), lambda i: (0, 0)),
        pl.BlockSpec((1, _D), lambda i: (0, 0)),
    ],
    out_specs=pl.BlockSpec((_R, _D), lambda i: (i, 0)),
    out_shape=jax.ShapeDtypeStruct((_NP, _D), jnp.float32),
)


def _tc_final_body(scale_ref, h_ref, p_ref, w4_ref, b4_ref, wn_ref, bn_ref,
                   wo_ref, bo_ref, bat_ref, o_ref, sums_ref, cnt_ref):
    i = pl.program_id(0)

    @pl.when(i == 0)
    def _():
        sums_ref[...] = jnp.zeros((_G, _D), jnp.float32)
        cnt_ref[...] = jnp.zeros((_G, _D), jnp.float32)

    z = scale_ref[0, 0] * h_ref[...] + p_ref[0] + p_ref[1]
    h5 = jnp.maximum(_dot(z, w4_ref[...], ((1,), (0,))) + b4_ref[...], 0.0)
    h6 = jnp.maximum(_dot(h5, wn_ref[...], ((1,), (0,))) + bn_ref[...], 0.0)
    gids = lax.broadcasted_iota(jnp.int32, (_R, _G), 1).astype(jnp.float32)
    oh = (bat_ref[...] == gids)
    oh = oh.astype(jnp.float32)
    sums_ref[...] += _dot(oh, h6, ((0,), (0,)))
    cnt_ref[...] += jnp.sum(oh, axis=0)[:, None]

    @pl.when(i == _NBLK - 1)
    def _():
        hg = sums_ref[...] / jnp.maximum(cnt_ref[...], 1.0)
        o_ref[...] = _dot(hg, wo_ref[...], ((1,), (0,))) + bo_ref[...]


_tc_final = pl.pallas_call(
    _tc_final_body,
    grid=(_NBLK,),
    in_specs=[
        pl.BlockSpec(memory_space=pltpu.SMEM),
        pl.BlockSpec((_R, _D), lambda i: (i, 0)),
        pl.BlockSpec((_NC, _R, _D), lambda i: (0, i, 0)),
        pl.BlockSpec((_D, _D), lambda i: (0, 0)),
        pl.BlockSpec((1, _D), lambda i: (0, 0)),
        pl.BlockSpec((_D, _D), lambda i: (0, 0)),
        pl.BlockSpec((1, _D), lambda i: (0, 0)),
        pl.BlockSpec((_D, _T), lambda i: (0, 0)),
        pl.BlockSpec((1, _T), lambda i: (0, 0)),
        pl.BlockSpec((_R, 1), lambda i: (i, 0)),
    ],
    out_specs=pl.BlockSpec((_G, _T), lambda i: (0, 0)),
    out_shape=jax.ShapeDtypeStruct((_G, _T), jnp.float32),
    scratch_shapes=[
        pltpu.VMEM((_G, _D), jnp.float32),
        pltpu.VMEM((_G, _D), jnp.float32),
    ],
)


def kernel(x, edge_index, batch, Wl, bl, eps, Wn2n, bn2n, Wout, bout):
    pad = _EPW - _E // _NW
    dummy_src = _N + (jnp.arange(pad, dtype=jnp.int32) % (_NP - _N))
    w_ids = jnp.arange(_NW, dtype=jnp.int32)[:, None]
    dummy_dst = (w_ids * 997 + jnp.arange(pad, dtype=jnp.int32)[None, :] * 131) % _NP
    src_b = jnp.concatenate(
        [edge_index[0].reshape(_NW, _E // _NW),
         jnp.broadcast_to(dummy_src, (_NW, pad))], axis=1).reshape(_NW, _NB, _K)
    dst_b = jnp.concatenate(
        [edge_index[1].reshape(_NW, _E // _NW),
         dummy_dst], axis=1).reshape(_NW, _NB, _K)
    batch_f = jnp.concatenate(
        [batch.astype(jnp.float32),
         jnp.full((_NP - _N,), -1.0, jnp.float32)]).reshape(_NP, 1)
    h = jnp.concatenate([x, jnp.zeros((_NP - _N, _D), jnp.float32)], axis=0)
    for l in range(_L):
        parts = _sc_edge_agg()(h, src_b, dst_b)
        scale = (1.0 + eps[l]).reshape(1, 1)
        b_l = bl[l].reshape(1, _D)
        if l < _L - 1:
            h = _tc_layer(scale, h, parts, Wl[l], b_l)
        else:
            out = _tc_final(scale, h, parts, Wl[l], b_l,
                            Wn2n, bn2n.reshape(1, _D),
                            Wout, bout.reshape(1, _T), batch_f)
    return out
